# baseline (speedup 1.0000x reference)
import functools

import jax
import jax.numpy as jnp
from jax.experimental import pallas as pl
from jax.experimental.pallas import tpu as pltpu

_B, _S, _IN_F, _OUT_F, _BANK, _K = 4, 2048, 1024, 1024, 16, 2
_H = _S // 2


def _body(sel_ref, p_ref, xa_ref, xb_ref, w0_ref, w1_ref, b0_ref, b1_ref, o_ref):
    b = pl.program_id(0)
    p0 = p_ref[b, 0]
    p1 = p_ref[b, 1]
    w = p0 * w0_ref[0] + p1 * w1_ref[0]
    bias = p0 * b0_ref[0] + p1 * b1_ref[0]
    acc_a = jnp.dot(xa_ref[0], w, preferred_element_type=jnp.float32)
    o_ref[0, : _H] = acc_a + bias
    acc_b = jnp.dot(xb_ref[0], w, preferred_element_type=jnp.float32)
    o_ref[0, _H :] = acc_b + bias


def kernel(x, selection_index, selection_probabilities, W_bank, b_bank):
    sel = selection_index.astype(jnp.int32)
    p = selection_probabilities.astype(jnp.float32)
    b3 = b_bank.reshape(_BANK, 1, _OUT_F)

    grid_spec = pltpu.PrefetchScalarGridSpec(
        num_scalar_prefetch=2,
        grid=(_B,),
        in_specs=[
            pl.BlockSpec((1, _H, _IN_F), lambda b, sel, p: (b, 0, 0)),
            pl.BlockSpec((1, _H, _IN_F), lambda b, sel, p: (b, 1, 0)),
            pl.BlockSpec((1, _IN_F, _OUT_F), lambda b, sel, p: (sel[b, 0], 0, 0)),
            pl.BlockSpec((1, _IN_F, _OUT_F), lambda b, sel, p: (sel[b, 1], 0, 0)),
            pl.BlockSpec((1, 1, _OUT_F), lambda b, sel, p: (sel[b, 0], 0, 0)),
            pl.BlockSpec((1, 1, _OUT_F), lambda b, sel, p: (sel[b, 1], 0, 0)),
        ],
        out_specs=pl.BlockSpec((1, _S, _OUT_F), lambda b, sel, p: (b, 0, 0)),
    )

    return pl.pallas_call(
        _body,
        grid_spec=grid_spec,
        out_shape=jax.ShapeDtypeStruct((_B, _S, _OUT_F), jnp.float32),
    )(sel, p, x, x, W_bank, W_bank, b3, b3)
